# baseline (speedup 1.0000x reference)
import functools
import math

import jax
import jax.numpy as jnp
import numpy as np
from jax import lax
from jax.experimental import pallas as pl
from jax.experimental.pallas import tpu as pltpu
from jax.experimental.pallas import tpu_sc as plsc

VOCAB = 100000
EMB = 126
D = 128
B = 1024
L = 200
NC = 2
NS = 16
NW = NC * NS
BPW = B // NW
SCALE = math.sqrt(float(D))
GCH = 40
NPB = L // GCH
NU = BPW * NPB
NBUF = 4


def _enc_const() -> np.ndarray:
    position = np.arange(L, dtype=np.float32)[:, None]
    div_term = np.exp(np.arange(0, D, 2, dtype=np.float32) * (-math.log(10000.0) / D))
    enc = np.zeros((L, D), dtype=np.float32)
    enc[:, 0::2] = np.sin(position * div_term)
    enc[:, 1::2] = np.cos(position * div_term)
    return enc


_MESH = plsc.VectorSubcoreMesh(core_axis_name="c", subcore_axis_name="s")


@functools.partial(
    pl.kernel,
    mesh=_MESH,
    out_type=jax.ShapeDtypeStruct((B, L, D), jnp.float32),
    scratch_types=(
        [pltpu.VMEM((BPW, NPB, GCH), jnp.int32),
         pltpu.VMEM((L, D), jnp.float32)]
        + [pltpu.VMEM((GCH, D), jnp.float32)] * NBUF
        + [pltpu.VMEM((GCH, 16), jnp.float32)] * NBUF
        + [pltpu.SemaphoreType.DMA] * (2 * NBUF)
    ),
)
def _sc_embed(tab_hbm, idx_hbm, thick_hbm, enc_hbm, out_hbm,
              idx_v, enc_v, *ring):
    ebufs = ring[:NBUF]
    tbufs = ring[NBUF:2 * NBUF]
    sgs = ring[2 * NBUF:3 * NBUF]
    sss = ring[3 * NBUF:4 * NBUF]

    wid = lax.axis_index("s") * NC + lax.axis_index("c")
    pltpu.sync_copy(idx_hbm.at[wid], idx_v)
    pltpu.sync_copy(enc_hbm, enc_v)

    def gather_descs(uu, fifth, p):
        return (
            pltpu.make_async_copy(tab_hbm.at[idx_v.at[uu, fifth]], ebufs[p], sgs[p]),
            pltpu.make_async_copy(thick_hbm.at[wid, uu, fifth], tbufs[p], sgs[p]),
        )

    def store_desc(uu, fifth, p):
        b = wid * BPW + uu
        off = pl.multiple_of(fifth * GCH, GCH)
        return pltpu.make_async_copy(
            ebufs[p], out_hbm.at[b, pl.ds(off, GCH)], sss[p])

    def compute(p, eb):
        ebuf, tbuf = ebufs[p], tbufs[p]

        def row_body(r, c):
            er = eb + r
            for k in range(D // 16 - 1):
                sl = pl.ds(k * 16, 16)
                ebuf[r, sl] = ebuf[r, sl] * SCALE + enc_v[er, sl]
            sl = pl.ds(D - 16, 16)
            ebuf[r, sl] = (ebuf[r, sl] + tbuf[r, :]) * SCALE + enc_v[er, sl]
            return c

        lax.fori_loop(0, GCH, row_body, 0)

    for d in gather_descs(0, 0, 0):
        d.start()

    def u4_body(u4, carry):
        for j in range(NBUF):
            u = NBUF * u4 + j
            pn = (j + 1) % NBUF

            @pl.when(u >= 3)
            def _():
                us = u - 3
                store_desc(us // NPB, us % NPB, pn).wait()

            @pl.when(u + 1 < NU)
            def _():
                un = u + 1
                for d in gather_descs(un // NPB, un % NPB, pn):
                    d.start()

            for d in gather_descs(u // NPB, u % NPB, j):
                d.wait()
            compute(j, (u % NPB) * GCH)
            store_desc(u // NPB, u % NPB, j).start()
        return carry

    lax.fori_loop(0, NU // NBUF, u4_body, 0)
    for us in range(NU - 3, NU):
        store_desc(us // NPB, us % NPB, us % NBUF).wait()


def kernel(x, table):
    idx = x[:, :, 0].astype(jnp.int32).reshape(NW, BPW, NPB, GCH)
    thick16 = jnp.pad(x[:, :, 1:], ((0, 0), (0, 0), (14, 0)))
    thick16 = thick16.reshape(NW, BPW, NPB, GCH, 16)
    tab = jnp.pad(table, ((0, 0), (0, D - EMB)))
    enc = jnp.asarray(_enc_const())
    return _sc_embed(tab, idx, thick16, enc)


# speedup vs baseline: 1.3579x; 1.3579x over previous
import functools
import math

import jax
import jax.numpy as jnp
import numpy as np
from jax import lax
from jax.experimental import pallas as pl
from jax.experimental.pallas import tpu as pltpu
from jax.experimental.pallas import tpu_sc as plsc

VOCAB = 100000
EMB = 126
D = 128
B = 1024
L = 200
NC = 2
NS = 16
NW = NC * NS
BPW = B // NW
SCALE = math.sqrt(float(D))
GCH = 100
NBUF = 3


def _enc_const() -> np.ndarray:
    position = np.arange(L, dtype=np.float32)[:, None]
    div_term = np.exp(np.arange(0, D, 2, dtype=np.float32) * (-math.log(10000.0) / D))
    enc = np.zeros((L, D), dtype=np.float32)
    enc[:, 0::2] = np.sin(position * div_term)
    enc[:, 1::2] = np.cos(position * div_term)
    return enc


_MESH = plsc.VectorSubcoreMesh(core_axis_name="c", subcore_axis_name="s")


@functools.partial(
    pl.kernel,
    mesh=_MESH,
    out_type=jax.ShapeDtypeStruct((B, L, D), jnp.float32),
    scratch_types=(
        [pltpu.VMEM((L, D), jnp.float32)]
        + [pltpu.VMEM((L, D), jnp.float32)] * NBUF
        + [pltpu.VMEM((2, GCH), jnp.int32)] * NBUF
        + [pltpu.VMEM((L, 16), jnp.float32)]
        + [pltpu.SemaphoreType.DMA] * (3 * NBUF)
    ),
)
def _sc_embed(tab_hbm, idx_hbm, thick_hbm, enc_hbm, out_hbm,
              enc_v, *ring):
    ebufs = ring[:NBUF]
    ibufs = ring[NBUF:2 * NBUF]
    thick_v = ring[2 * NBUF]
    sgs = ring[2 * NBUF + 1:2 * NBUF + 1 + NBUF]
    sss = ring[2 * NBUF + 1 + NBUF:2 * NBUF + 1 + 2 * NBUF]
    sis = ring[2 * NBUF + 1 + 2 * NBUF:]

    wid = lax.axis_index("s") * NC + lax.axis_index("c")
    pltpu.sync_copy(enc_hbm, enc_v)

    def idx_desc(bb, p):
        return pltpu.make_async_copy(idx_hbm.at[wid, bb], ibufs[p], sis[p])

    def gather_descs(bb, p):
        return (
            pltpu.make_async_copy(tab_hbm.at[ibufs[p].at[0]],
                                  ebufs[p].at[pl.ds(0, GCH)], sgs[p]),
            pltpu.make_async_copy(tab_hbm.at[ibufs[p].at[1]],
                                  ebufs[p].at[pl.ds(GCH, GCH)], sgs[p]),
        )

    def store_desc(bb, p):
        return pltpu.make_async_copy(ebufs[p], out_hbm.at[wid * BPW + bb], sss[p])

    def compute(p):
        ebuf, tbuf = ebufs[p], thick_v

        def row_body(r, c):
            for k in range(D // 16 - 1):
                sl = pl.ds(k * 16, 16)
                ebuf[r, sl] = ebuf[r, sl] * SCALE + enc_v[r, sl]
            sl = pl.ds(D - 16, 16)
            ebuf[r, sl] = (ebuf[r, sl] + tbuf[r, :]) * SCALE + enc_v[r, sl]
            return c

        lax.fori_loop(0, L, row_body, 0)

    def unit(bb, j, last):
        pn = (j + 1) % NBUF
        pnn = (j + 2) % NBUF
        if isinstance(bb, int):
            if bb >= 2:
                store_desc(bb - 2, pn).wait()
        else:
            @pl.when(bb >= 2)
            def _():
                store_desc(bb - 2, pn).wait()
        if not last:
            idx_desc(bb + 1, pn).wait()
            for d in gather_descs(bb + 1, pn):
                d.start()
            if isinstance(bb, int):
                if bb + 2 < BPW:
                    idx_desc(bb + 2, pnn).start()
            else:
                @pl.when(bb + 2 < BPW)
                def _():
                    idx_desc(bb + 2, pnn).start()
        for d in gather_descs(bb, j):
            d.wait()
        pltpu.sync_copy(thick_hbm.at[wid, bb], thick_v)
        compute(j)
        store_desc(bb, j).start()

    pltpu.sync_copy(idx_hbm.at[wid, 0], ibufs[0])
    idx_desc(1, 1).start()
    for d in gather_descs(0, 0):
        d.start()
    unit(0, 0, False)

    def t_body(t, carry):
        for j in range(NBUF):
            bb = NBUF * t + j + 1
            unit(bb, (j + 1) % NBUF, False)
        return carry

    lax.fori_loop(0, (BPW - 2) // NBUF, t_body, 0)
    unit(BPW - 1, (BPW - 1) % NBUF, True)
    store_desc(BPW - 2, (BPW - 2) % NBUF).wait()
    store_desc(BPW - 1, (BPW - 1) % NBUF).wait()


def kernel(x, table):
    idx = x[:, :, 0].astype(jnp.int32).reshape(NW, BPW, 2, GCH)
    thick16 = jnp.pad(x[:, :, 1:], ((0, 0), (0, 0), (14, 0)))
    thick16 = thick16.reshape(NW, BPW, L, 16)
    tab = jnp.pad(table, ((0, 0), (0, D - EMB)))
    enc = jnp.asarray(_enc_const())
    return _sc_embed(tab, idx, thick16, enc)


# speedup vs baseline: 1.8191x; 1.3397x over previous
import functools
import math

import jax
import jax.numpy as jnp
import numpy as np
from jax import lax
from jax.experimental import pallas as pl
from jax.experimental.pallas import tpu as pltpu
from jax.experimental.pallas import tpu_sc as plsc

VOCAB = 100000
EMB = 126
D = 128
B = 1024
L = 200
NC = 2
NS = 16
NW = NC * NS
BPW = B // NW
SCALE = math.sqrt(float(D))
GCH = 100
NBUF = 3


def _enc_const() -> np.ndarray:
    position = np.arange(L, dtype=np.float32)[:, None]
    div_term = np.exp(np.arange(0, D, 2, dtype=np.float32) * (-math.log(10000.0) / D))
    enc = np.zeros((L, D), dtype=np.float32)
    enc[:, 0::2] = np.sin(position * div_term)
    enc[:, 1::2] = np.cos(position * div_term)
    return enc


_MESH = plsc.VectorSubcoreMesh(core_axis_name="c", subcore_axis_name="s")


@functools.partial(
    pl.kernel,
    mesh=_MESH,
    out_type=jax.ShapeDtypeStruct((B, L, D), jnp.float32),
    scratch_types=(
        [pltpu.VMEM((L, D), jnp.float32)]
        + [pltpu.VMEM((L, D), jnp.float32)] * NBUF
        + [pltpu.VMEM((2, GCH), jnp.int32)] * NBUF
        + [pltpu.VMEM((L * 16,), jnp.float32)]
        + [pltpu.SemaphoreType.DMA] * (3 * NBUF)
    ),
)
def _sc_embed(tab_hbm, idx_hbm, thick_hbm, enc_hbm, out_hbm,
              enc_v, *ring):
    ebufs = ring[:NBUF]
    ibufs = ring[NBUF:2 * NBUF]
    thick_v = ring[2 * NBUF]
    sgs = ring[2 * NBUF + 1:2 * NBUF + 1 + NBUF]
    sss = ring[2 * NBUF + 1 + NBUF:2 * NBUF + 1 + 2 * NBUF]
    sis = ring[2 * NBUF + 1 + 2 * NBUF:]

    wid = lax.axis_index("s") * NC + lax.axis_index("c")
    pltpu.sync_copy(enc_hbm, enc_v)

    def idx_desc(bb, p):
        return pltpu.make_async_copy(idx_hbm.at[wid, bb], ibufs[p], sis[p])

    def gather_descs(bb, p):
        return (
            pltpu.make_async_copy(tab_hbm.at[ibufs[p].at[0]],
                                  ebufs[p].at[pl.ds(0, GCH)], sgs[p]),
            pltpu.make_async_copy(tab_hbm.at[ibufs[p].at[1]],
                                  ebufs[p].at[pl.ds(GCH, GCH)], sgs[p]),
        )

    def store_desc(bb, p):
        return pltpu.make_async_copy(ebufs[p], out_hbm.at[wid * BPW + bb], sss[p])

    def compute(p):
        ebuf, tbuf = ebufs[p], thick_v

        def row_body(r, c):
            for k in range(D // 16 - 1):
                sl = pl.ds(k * 16, 16)
                ebuf[r, sl] = ebuf[r, sl] * SCALE + enc_v[r, sl]
            sl = pl.ds(D - 16, 16)
            tb = tbuf[pl.ds(r * 16, 16)]
            ebuf[r, sl] = (ebuf[r, sl] + tb) * SCALE + enc_v[r, sl]
            return c

        lax.fori_loop(0, L, row_body, 0)

    def unit(bb, j, last):
        pn = (j + 1) % NBUF
        pnn = (j + 2) % NBUF
        if isinstance(bb, int):
            if bb >= 2:
                store_desc(bb - 2, pn).wait()
        else:
            @pl.when(bb >= 2)
            def _():
                store_desc(bb - 2, pn).wait()
        if not last:
            idx_desc(bb + 1, pn).wait()
            for d in gather_descs(bb + 1, pn):
                d.start()
            if isinstance(bb, int):
                if bb + 2 < BPW:
                    idx_desc(bb + 2, pnn).start()
            else:
                @pl.when(bb + 2 < BPW)
                def _():
                    idx_desc(bb + 2, pnn).start()
        for d in gather_descs(bb, j):
            d.wait()
        pltpu.sync_copy(thick_hbm.at[wid, bb], thick_v)
        compute(j)
        store_desc(bb, j).start()

    pltpu.sync_copy(idx_hbm.at[wid, 0], ibufs[0])
    idx_desc(1, 1).start()
    for d in gather_descs(0, 0):
        d.start()
    unit(0, 0, False)

    def t_body(t, carry):
        for j in range(NBUF):
            bb = NBUF * t + j + 1
            unit(bb, (j + 1) % NBUF, False)
        return carry

    lax.fori_loop(0, (BPW - 2) // NBUF, t_body, 0)
    unit(BPW - 1, (BPW - 1) % NBUF, True)
    store_desc(BPW - 2, (BPW - 2) % NBUF).wait()
    store_desc(BPW - 1, (BPW - 1) % NBUF).wait()


def kernel(x, table):
    idx = x[:, :, 0].astype(jnp.int32).reshape(NW, BPW, 2, GCH)
    thick16 = jnp.pad(x[:, :, 1:], ((0, 0), (0, 0), (14, 0)))
    thick16 = thick16.reshape(NW, BPW, L * 16)
    tab = jnp.pad(table, ((0, 0), (0, D - EMB)))
    enc = jnp.asarray(_enc_const())
    return _sc_embed(tab, idx, thick16, enc)


# speedup vs baseline: 2.8286x; 1.5549x over previous
import functools
import math

import jax
import jax.numpy as jnp
import numpy as np
from jax import lax
from jax.experimental import pallas as pl
from jax.experimental.pallas import tpu as pltpu
from jax.experimental.pallas import tpu_sc as plsc

VOCAB = 100000
EMB = 126
D = 128
B = 1024
L = 200
NC = 2
NS = 16
NW = NC * NS
BPW = B // NW
SCALE = math.sqrt(float(D))
GCH = 100
NBUF = 3


def _enc_const() -> np.ndarray:
    position = np.arange(L, dtype=np.float32)[:, None]
    div_term = np.exp(np.arange(0, D, 2, dtype=np.float32) * (-math.log(10000.0) / D))
    enc = np.zeros((L, D), dtype=np.float32)
    enc[:, 0::2] = np.sin(position * div_term)
    enc[:, 1::2] = np.cos(position * div_term)
    return enc


_MESH = plsc.VectorSubcoreMesh(core_axis_name="c", subcore_axis_name="s")


@functools.partial(
    pl.kernel,
    mesh=_MESH,
    out_type=jax.ShapeDtypeStruct((B, L, D), jnp.float32),
    scratch_types=(
        [pltpu.VMEM((L, D), jnp.float32)]
        + [pltpu.VMEM((L, D), jnp.float32)] * NBUF
        + [pltpu.VMEM((2, GCH), jnp.int32)] * NBUF
        + [pltpu.VMEM((2 * L + 16,), jnp.float32)]
        + [pltpu.SemaphoreType.DMA] * (3 * NBUF)
    ),
)
def _sc_embed(tab_hbm, idx_hbm, thick_hbm, enc_hbm, out_hbm,
              enc_v, *ring):
    ebufs = ring[:NBUF]
    ibufs = ring[NBUF:2 * NBUF]
    thick_v = ring[2 * NBUF]
    sgs = ring[2 * NBUF + 1:2 * NBUF + 1 + NBUF]
    sss = ring[2 * NBUF + 1 + NBUF:2 * NBUF + 1 + 2 * NBUF]
    sis = ring[2 * NBUF + 1 + 2 * NBUF:]

    wid = lax.axis_index("s") * NC + lax.axis_index("c")
    pltpu.sync_copy(enc_hbm, enc_v)
    tail_lane = lax.iota(jnp.int32, 16) >= 14

    def idx_desc(bb, p):
        return pltpu.make_async_copy(idx_hbm.at[wid, bb], ibufs[p], sis[p])

    def gather_descs(bb, p):
        return (
            pltpu.make_async_copy(tab_hbm.at[ibufs[p].at[0]],
                                  ebufs[p].at[pl.ds(0, GCH)], sgs[p]),
            pltpu.make_async_copy(tab_hbm.at[ibufs[p].at[1]],
                                  ebufs[p].at[pl.ds(GCH, GCH)], sgs[p]),
        )

    def store_desc(bb, p):
        return pltpu.make_async_copy(ebufs[p], out_hbm.at[wid * BPW + bb], sss[p])

    def compute(p):
        ebuf, tbuf = ebufs[p], thick_v

        def row_body(r, c):
            for k in range(D // 16 - 1):
                sl = pl.ds(k * 16, 16)
                ebuf[r, sl] = ebuf[r, sl] * SCALE + enc_v[r, sl]
            sl = pl.ds(D - 16, 16)
            tb = tbuf[pl.ds(2 * r + 2, 16)]
            tb = jnp.where(tail_lane, tb, 0.0)
            ebuf[r, sl] = (ebuf[r, sl] + tb) * SCALE + enc_v[r, sl]
            return c

        lax.fori_loop(0, L, row_body, 0)

    def unit(bb, j, last):
        pn = (j + 1) % NBUF
        pnn = (j + 2) % NBUF
        if isinstance(bb, int):
            if bb >= 2:
                store_desc(bb - 2, pn).wait()
        else:
            @pl.when(bb >= 2)
            def _():
                store_desc(bb - 2, pn).wait()
        if not last:
            idx_desc(bb + 1, pn).wait()
            for d in gather_descs(bb + 1, pn):
                d.start()
            if isinstance(bb, int):
                if bb + 2 < BPW:
                    idx_desc(bb + 2, pnn).start()
            else:
                @pl.when(bb + 2 < BPW)
                def _():
                    idx_desc(bb + 2, pnn).start()
        for d in gather_descs(bb, j):
            d.wait()
        pltpu.sync_copy(thick_hbm.at[wid, bb], thick_v)
        compute(j)
        store_desc(bb, j).start()

    pltpu.sync_copy(idx_hbm.at[wid, 0], ibufs[0])
    idx_desc(1, 1).start()
    for d in gather_descs(0, 0):
        d.start()
    unit(0, 0, False)

    def t_body(t, carry):
        for j in range(NBUF):
            bb = NBUF * t + j + 1
            unit(bb, (j + 1) % NBUF, False)
        return carry

    lax.fori_loop(0, (BPW - 2) // NBUF, t_body, 0)
    unit(BPW - 1, (BPW - 1) % NBUF, True)
    store_desc(BPW - 2, (BPW - 2) % NBUF).wait()
    store_desc(BPW - 1, (BPW - 1) % NBUF).wait()


def kernel(x, table):
    idx = x[:, :, 0].astype(jnp.int32).reshape(NW, BPW, 2, GCH)
    thick16 = jnp.pad(x[:, :, 1:].reshape(B, 2 * L), ((0, 0), (16, 0)))
    thick16 = thick16.reshape(NW, BPW, 2 * L + 16)
    tab = jnp.pad(table, ((0, 0), (0, D - EMB)))
    enc = jnp.asarray(_enc_const())
    return _sc_embed(tab, idx, thick16, enc)


# speedup vs baseline: 3.1594x; 1.1169x over previous
import functools
import math

import jax
import jax.numpy as jnp
import numpy as np
from jax import lax
from jax.experimental import pallas as pl
from jax.experimental.pallas import tpu as pltpu
from jax.experimental.pallas import tpu_sc as plsc

VOCAB = 100000
EMB = 126
D = 128
B = 1024
L = 200
NC = 2
NS = 16
NW = NC * NS
BPW = B // NW
SCALE = math.sqrt(float(D))
GCH = 100
NBUF = 3
TW = 2 * L + 16


def _enc_const() -> np.ndarray:
    position = np.arange(L, dtype=np.float32)[:, None]
    div_term = np.exp(np.arange(0, D, 2, dtype=np.float32) * (-math.log(10000.0) / D))
    enc = np.zeros((L, D), dtype=np.float32)
    enc[:, 0::2] = np.sin(position * div_term)
    enc[:, 1::2] = np.cos(position * div_term)
    return enc


_MESH = plsc.VectorSubcoreMesh(core_axis_name="c", subcore_axis_name="s")


@functools.partial(
    pl.kernel,
    mesh=_MESH,
    out_type=jax.ShapeDtypeStruct((B, L, D), jnp.float32),
    scratch_types=(
        [pltpu.VMEM((L, D), jnp.float32)]
        + [pltpu.VMEM((L, D), jnp.float32)] * NBUF
        + [pltpu.VMEM((2, GCH), jnp.int32)] * NBUF
        + [pltpu.VMEM((TW,), jnp.float32)] * NBUF
        + [pltpu.SemaphoreType.DMA] * (3 * NBUF)
    ),
)
def _sc_embed(tab_hbm, idx_hbm, thick_hbm, enc_hbm, out_hbm,
              enc_v, *ring):
    ebufs = ring[:NBUF]
    ibufs = ring[NBUF:2 * NBUF]
    tbufs = ring[2 * NBUF:3 * NBUF]
    sgs = ring[3 * NBUF:4 * NBUF]
    sss = ring[4 * NBUF:5 * NBUF]
    sis = ring[5 * NBUF:]

    wid = lax.axis_index("s") * NC + lax.axis_index("c")
    pltpu.sync_copy(enc_hbm, enc_v)
    tail_lane = lax.iota(jnp.int32, 16) >= 14

    def idx_desc(bb, p):
        return pltpu.make_async_copy(idx_hbm.at[wid, bb], ibufs[p], sis[p])

    def gather_descs(bb, p):
        return (
            pltpu.make_async_copy(tab_hbm.at[ibufs[p].at[0]],
                                  ebufs[p].at[pl.ds(0, GCH)], sgs[p]),
            pltpu.make_async_copy(tab_hbm.at[ibufs[p].at[1]],
                                  ebufs[p].at[pl.ds(GCH, GCH)], sgs[p]),
            pltpu.make_async_copy(thick_hbm.at[wid, bb], tbufs[p], sgs[p]),
        )

    def store_desc(bb, p):
        return pltpu.make_async_copy(ebufs[p], out_hbm.at[wid * BPW + bb], sss[p])

    def compute(p):
        ebuf, tbuf = ebufs[p], tbufs[p]

        def row_body(r, c):
            for k in range(D // 16 - 1):
                sl = pl.ds(k * 16, 16)
                ebuf[r, sl] = ebuf[r, sl] * SCALE + enc_v[r, sl]
            sl = pl.ds(D - 16, 16)
            tb = tbuf[pl.ds(2 * r + 2, 16)]
            tb = jnp.where(tail_lane, tb, 0.0)
            ebuf[r, sl] = (ebuf[r, sl] + tb) * SCALE + enc_v[r, sl]
            return c

        lax.fori_loop(0, L, row_body, 0)

    def unit(bb, j, last):
        pn = (j + 1) % NBUF
        pnn = (j + 2) % NBUF
        if isinstance(bb, int):
            if bb >= 2:
                store_desc(bb - 2, pn).wait()
        else:
            @pl.when(bb >= 2)
            def _():
                store_desc(bb - 2, pn).wait()
        if not last:
            idx_desc(bb + 1, pn).wait()
            for d in gather_descs(bb + 1, pn):
                d.start()
            if isinstance(bb, int):
                if bb + 2 < BPW:
                    idx_desc(bb + 2, pnn).start()
            else:
                @pl.when(bb + 2 < BPW)
                def _():
                    idx_desc(bb + 2, pnn).start()
        for d in gather_descs(bb, j):
            d.wait()
        compute(j)
        store_desc(bb, j).start()

    pltpu.sync_copy(idx_hbm.at[wid, 0], ibufs[0])
    idx_desc(1, 1).start()
    for d in gather_descs(0, 0):
        d.start()
    unit(0, 0, False)

    def t_body(t, carry):
        for j in range(NBUF):
            bb = NBUF * t + j + 1
            unit(bb, (j + 1) % NBUF, False)
        return carry

    lax.fori_loop(0, (BPW - 2) // NBUF, t_body, 0)
    unit(BPW - 1, (BPW - 1) % NBUF, True)
    store_desc(BPW - 2, (BPW - 2) % NBUF).wait()
    store_desc(BPW - 1, (BPW - 1) % NBUF).wait()


def kernel(x, table):
    idx = x[:, :, 0].astype(jnp.int32).reshape(NW, BPW, 2, GCH)
    thick = jnp.pad(x[:, :, 1:].reshape(B, 2 * L), ((0, 0), (16, 0)))
    thick = thick.reshape(NW, BPW, TW)
    tab = jnp.pad(table, ((0, 0), (0, D - EMB)))
    enc = jnp.asarray(_enc_const())
    return _sc_embed(tab, idx, thick, enc)
